# Initial kernel scaffold; baseline (speedup 1.0000x reference)
import jax
import jax.numpy as jnp
from jax.experimental import pallas as pl


def kernel(nodes, edges, senders, receivers, W_e, b_e, W_n, b_n, gamma_n, beta_n, gamma_e, beta_e):
    raise NotImplementedError("write your pallas kernel here")


# speedup vs baseline: 2.7084x; 2.7084x over previous
import jax
import jax.numpy as jnp
from jax import lax
from jax.experimental import pallas as pl
from jax.experimental.pallas import tpu as pltpu
from jax.experimental.pallas import tpu_sc as plsc

NC = 2
NS = 16
NW = NC * NS
CH = 128


def _pre_body(x_ref, w_ref, b_ref, t2_ref, t3_ref, u_ref):
    d = t2_ref.shape[-1]
    t = jnp.dot(x_ref[...], w_ref[...], preferred_element_type=jnp.float32)
    t = t + b_ref[...]
    t2_ref[...] = t[:, :d]
    t3_ref[...] = t[:, d:2 * d]
    u_ref[...] = t[:, 2 * d:]


def _ln(x, gamma, beta):
    mean = jnp.mean(x, axis=-1, keepdims=True)
    xc = x - mean
    var = jnp.mean(xc * xc, axis=-1, keepdims=True)
    return xc * lax.rsqrt(var + 1e-6) * gamma + beta


def _edge_body(e_ref, g_ref, w_ref, gam_ref, bet_ref, ne_ref, eo_ref):
    e = e_ref[...]
    ne = jnp.dot(e, w_ref[...], preferred_element_type=jnp.float32) + g_ref[...]
    ne_ref[...] = ne
    eo_ref[...] = _ln(ne + e, gam_ref[...], bet_ref[...])


def _node_body(u_ref, p_ref, x_ref, w_ref, gam_ref, bet_ref, o_ref):
    received = p_ref[0] + p_ref[1]
    nn = u_ref[...] + jnp.dot(received, w_ref[...],
                              preferred_element_type=jnp.float32)
    o_ref[...] = _ln(nn + x_ref[...], gam_ref[...], bet_ref[...])


def _gather_body(t2_hbm, t3_hbm, snd_hbm, rcv_hbm, g_hbm,
                 sidx, ridx, abuf, bbuf, sem1, sem2):
    c = lax.axis_index("c")
    s = lax.axis_index("s")
    wid = s * NC + c
    nchunk = g_hbm.shape[0] // CH
    kmax = (nchunk - wid + NW - 1) // NW

    def body(k, carry):
        base = (k * NW + wid) * CH
        pltpu.sync_copy(snd_hbm.at[pl.ds(base, CH)], sidx)
        pltpu.sync_copy(rcv_hbm.at[pl.ds(base, CH)], ridx)
        cp1 = pltpu.async_copy(t2_hbm.at[sidx], abuf, sem1)
        cp2 = pltpu.async_copy(t3_hbm.at[ridx], bbuf, sem2)
        cp1.wait()
        cp2.wait()

        def add_row(r, carry2):
            for j in range(8):
                sl = pl.ds(j * 16, 16)
                abuf[r, sl] = abuf[r, sl] + bbuf[r, sl]
            return carry2

        lax.fori_loop(0, CH, add_row, 0, unroll=2)
        pltpu.sync_copy(abuf, g_hbm.at[pl.ds(base, CH)])
        return carry

    lax.fori_loop(0, kmax, body, 0)


def _scatter_body(ne_hbm, rcv_hbm, zero_hbm, p_hbm, ridx, rows, accum):
    c = lax.axis_index("c")
    s = lax.axis_index("s")
    wid = s * NC + c
    n = accum.shape[0]
    rows_per = n // NS
    pltpu.sync_copy(zero_hbm.at[pl.ds(s * rows_per, rows_per)],
                    accum.at[pl.ds(s * rows_per, rows_per)])
    plsc.subcore_barrier()

    nchunk = ne_hbm.shape[0] // CH
    kmax = (nchunk - wid + NW - 1) // NW

    def body(k, carry):
        base = (k * NW + wid) * CH
        pltpu.sync_copy(rcv_hbm.at[pl.ds(base, CH)], ridx)
        pltpu.sync_copy(ne_hbm.at[pl.ds(base, CH)], rows)
        pltpu.sync_copy(rows, accum.at[ridx], add=True)
        return carry

    lax.fori_loop(0, kmax, body, 0)
    plsc.subcore_barrier()
    pltpu.sync_copy(accum.at[pl.ds(s * rows_per, rows_per)],
                    p_hbm.at[c, pl.ds(s * rows_per, rows_per)])


def kernel(nodes, edges, senders, receivers, W_e, b_e, W_n, b_n,
           gamma_n, beta_n, gamma_e, beta_e):
    N, D = nodes.shape
    E = edges.shape[0]
    assert D == 128 and N % NS == 0 and E % CH == 0

    W1 = W_e[:D]
    Wcat = jnp.concatenate([W_e[D:2 * D], W_e[2 * D:], W_n[:D]], axis=1)
    bcat = jnp.concatenate(
        [b_e, jnp.zeros_like(b_e), b_n]).reshape(1, 3 * D)
    Wn2 = W_n[D:]
    gam_e = gamma_e.reshape(1, D)
    bet_e = beta_e.reshape(1, D)
    gam_n = gamma_n.reshape(1, D)
    bet_n = beta_n.reshape(1, D)

    BN = 2000
    t2, t3, u = pl.pallas_call(
        _pre_body,
        grid=(N // BN,),
        in_specs=[
            pl.BlockSpec((BN, D), lambda i: (i, 0)),
            pl.BlockSpec((D, 3 * D), lambda i: (0, 0)),
            pl.BlockSpec((1, 3 * D), lambda i: (0, 0)),
        ],
        out_specs=[pl.BlockSpec((BN, D), lambda i: (i, 0))] * 3,
        out_shape=[jax.ShapeDtypeStruct((N, D), jnp.float32)] * 3,
    )(nodes, Wcat, bcat)

    mesh = plsc.VectorSubcoreMesh(core_axis_name="c", subcore_axis_name="s",
                                  num_cores=NC, num_subcores=NS)
    g = pl.kernel(
        _gather_body,
        out_type=jax.ShapeDtypeStruct((E, D), jnp.float32),
        mesh=mesh,
        scratch_types=[
            pltpu.VMEM((CH,), jnp.int32),
            pltpu.VMEM((CH,), jnp.int32),
            pltpu.VMEM((CH, D), jnp.float32),
            pltpu.VMEM((CH, D), jnp.float32),
            pltpu.SemaphoreType.DMA,
            pltpu.SemaphoreType.DMA,
        ],
    )(t2, t3, senders, receivers)

    BE = 2000
    ne, eo = pl.pallas_call(
        _edge_body,
        grid=(E // BE,),
        in_specs=[
            pl.BlockSpec((BE, D), lambda i: (i, 0)),
            pl.BlockSpec((BE, D), lambda i: (i, 0)),
            pl.BlockSpec((D, D), lambda i: (0, 0)),
            pl.BlockSpec((1, D), lambda i: (0, 0)),
            pl.BlockSpec((1, D), lambda i: (0, 0)),
        ],
        out_specs=[pl.BlockSpec((BE, D), lambda i: (i, 0))] * 2,
        out_shape=[jax.ShapeDtypeStruct((E, D), jnp.float32)] * 2,
    )(edges, g, W1, gam_e, bet_e)

    n_pad = ((N + 8 * NS - 1) // (8 * NS)) * (8 * NS)
    zeros = jnp.zeros((n_pad, D), jnp.float32)
    p = pl.kernel(
        _scatter_body,
        out_type=jax.ShapeDtypeStruct((NC, n_pad, D), jnp.float32),
        mesh=mesh,
        scratch_types=[
            pltpu.VMEM((CH,), jnp.int32),
            pltpu.VMEM((CH, D), jnp.float32),
            pltpu.VMEM_SHARED((n_pad, D), jnp.float32),
        ],
    )(ne, receivers, zeros)

    nodes_out = pl.pallas_call(
        _node_body,
        grid=(N // BN,),
        in_specs=[
            pl.BlockSpec((BN, D), lambda i: (i, 0)),
            pl.BlockSpec((NC, BN, D), lambda i: (0, i, 0)),
            pl.BlockSpec((BN, D), lambda i: (i, 0)),
            pl.BlockSpec((D, D), lambda i: (0, 0)),
            pl.BlockSpec((1, D), lambda i: (0, 0)),
            pl.BlockSpec((1, D), lambda i: (0, 0)),
        ],
        out_specs=pl.BlockSpec((BN, D), lambda i: (i, 0)),
        out_shape=jax.ShapeDtypeStruct((N, D), jnp.float32),
    )(u, p, nodes, Wn2, gam_n, bet_n)

    return nodes_out, eo
